# baseline (speedup 1.0000x reference)
import functools

import jax
import jax.numpy as jnp
from jax import lax
from jax.experimental import pallas as pl
from jax.experimental.pallas import tpu as pltpu
from jax.experimental.pallas import tpu_sc as plsc

DIM = 128
NSLICE = DIM // 16
CHUNK = 32
NTILES = 32
BLK = 512


def _sc_edge_body(npt, nlog,
                  w_hbm, sp_hbm, rows_hbm, cols_hbm, off_hbm,
                  aggs_hbm, aggv_hbm,
                  off_v, sp0, sp1, w0, w1, wi0, wi1,
                  rows0, rows1, cols0, cols1,
                  accs, accv, semg0, semg1, semi0, semi1):
    wid = lax.axis_index("s") * 2 + lax.axis_index("c")
    base_node = wid * npt
    words = npt * DIM
    nmask = (1 << nlog) - 1

    pltpu.sync_copy(off_hbm, off_v)
    lo = off_v[pl.ds(wid, 16)][0]
    hi = off_v[pl.ds(wid + 1, 16)][0]
    start = jnp.bitwise_and(lo, jnp.int32(~7))
    nch = (hi - start + (CHUNK - 1)) // CHUNK
    npairs = (nch + 1) // 2

    zeros16 = jnp.zeros((16,), jnp.float32)

    def zbody(t, c):
        for u in range(4):
            accs[pl.ds(t * 64 + 16 * u, 16)] = zeros16
            accv[pl.ds(t * 64 + 16 * u, 16)] = zeros16
        return c

    lax.fori_loop(0, words // 64, zbody, 0)

    def ds8(x):
        return pl.ds(pl.multiple_of(x, 8), CHUNK)

    def drain(dst, sem):
        pltpu.make_async_copy(rows_hbm.at[pl.ds(0, CHUNK)], dst, sem).wait()

    def drain_sp(dst, sem):
        pltpu.make_async_copy(sp_hbm.at[pl.ds(0, CHUNK)], dst, sem).wait()

    def drain_w(dst, sem):
        pltpu.make_async_copy(w_hbm.at[pl.ds(0, CHUNK)], dst, sem).wait()

    def fire_gathers(rowsr, colsr, wir, wr, spr, sem):
        for h in range(CHUNK // 16):
            r16 = rowsr[pl.ds(16 * h, 16)]
            c16 = colsr[pl.ds(16 * h, 16)]
            widx = ((jnp.bitwise_and(r16, nmask) << nlog)
                    + jnp.bitwise_and(c16, nmask)
                    + ((r16 >> nlog) << (2 * nlog)))
            wir[pl.ds(16 * h, 16)] = widx
        pltpu.async_copy(w_hbm.at[wir], wr, sem)
        pltpu.async_copy(sp_hbm.at[colsr], spr, sem)

    lane = lax.broadcasted_iota(jnp.int32, (16,), 0)

    def compute(e0c, rvecs, wr, spr, carry):
        prev_rb, cs, cp = carry
        for h in range(CHUNK // 16):
            r16 = rvecs[h]
            w16 = wr[pl.ds(16 * h, 16)]
            eid16 = (e0c + 16 * h) + lane
            validm = jnp.where(
                jnp.logical_and(eid16 >= lo, eid16 < hi),
                jnp.float32(1.0), jnp.float32(0.0))
            wm = w16 * validm
            rloc = jnp.clip(r16 - base_node, 0, npt - 1) * DIM
            for e in range(16):
                ei = 16 * h + e
                rb_e = rloc[e]
                changed = rb_e != prev_rb

                @pl.when(changed)
                def _flush(prev_rb=prev_rb, cs=cs, cp=cp):
                    for j in range(NSLICE):
                        accs[pl.ds(prev_rb + 16 * j, 16)] = cs[j]
                        accv[pl.ds(prev_rb + 16 * j, 16)] = cp[j]

                keep = jnp.where(changed, jnp.float32(0.0),
                                 jnp.float32(1.0))
                sel = jnp.full((16,), e, jnp.int32)
                w_b = wm.at[sel].get(mode="promise_in_bounds")
                ncs, ncp = list(cs), list(cp)
                for u in range(2 * DIM // 32):
                    xi = spr[ei, pl.ds(16 * u, 16)]
                    vlo = lax.bitcast_convert_type(xi << 16, jnp.float32)
                    vhi = lax.bitcast_convert_type(
                        jnp.bitwise_and(xi, jnp.int32(-65536)),
                        jnp.float32)
                    if u < DIM // 32:
                        ncs[2 * u] = cs[2 * u] * keep + w_b * vlo
                        ncs[2 * u + 1] = cs[2 * u + 1] * keep + w_b * vhi
                    else:
                        t = u - DIM // 32
                        ncp[2 * t] = cp[2 * t] * keep + w_b * vlo
                        ncp[2 * t + 1] = cp[2 * t + 1] * keep + w_b * vhi
                cs, cp = tuple(ncs), tuple(ncp)
                prev_rb = rb_e
        return prev_rb, cs, cp

    pltpu.sync_copy(rows_hbm.at[ds8(start)], rows0)
    pltpu.sync_copy(cols_hbm.at[ds8(start)], cols0)
    fire_gathers(rows0, cols0, wi0, w0, sp0, semg0)
    pltpu.async_copy(rows_hbm.at[ds8(start + CHUNK)], rows1, semi1)
    pltpu.async_copy(cols_hbm.at[ds8(start + CHUNK)], cols1, semi1)

    def pbody(gg, carry):
        ca = start + (2 * gg) * CHUNK
        drain_w(w0, semg0)
        drain_sp(sp0, semg0)
        ra = [rows0[pl.ds(16 * h, 16)] for h in range(CHUNK // 16)]
        drain(rows1, semi1)
        drain(cols1, semi1)
        fire_gathers(rows1, cols1, wi1, w1, sp1, semg1)
        pltpu.async_copy(rows_hbm.at[ds8(ca + 2 * CHUNK)], rows0, semi0)
        pltpu.async_copy(cols_hbm.at[ds8(ca + 2 * CHUNK)], cols0, semi0)
        carry = compute(ca, ra, w0, sp0, carry)
        drain_w(w1, semg1)
        drain_sp(sp1, semg1)
        rb = [rows1[pl.ds(16 * h, 16)] for h in range(CHUNK // 16)]
        drain(rows0, semi0)
        drain(cols0, semi0)
        fire_gathers(rows0, cols0, wi0, w0, sp0, semg0)
        pltpu.async_copy(rows_hbm.at[ds8(ca + 3 * CHUNK)], rows1, semi1)
        pltpu.async_copy(cols_hbm.at[ds8(ca + 3 * CHUNK)], cols1, semi1)
        carry = compute(ca + CHUNK, rb, w1, sp1, carry)
        return carry

    zero8 = tuple(jnp.zeros((16,), jnp.float32) for _ in range(NSLICE))
    carry0 = (jnp.int32(0), zero8, zero8)
    prev_rb, cs, cp = lax.fori_loop(0, npairs, pbody, carry0)

    for j in range(NSLICE):
        accs[pl.ds(prev_rb + 16 * j, 16)] = cs[j]
        accv[pl.ds(prev_rb + 16 * j, 16)] = cp[j]

    drain_w(w0, semg0)
    drain_sp(sp0, semg0)
    drain(rows1, semi1)
    drain(cols1, semi1)

    pltpu.sync_copy(accs, aggs_hbm.at[pl.ds(wid * words, words)])
    pltpu.sync_copy(accv, aggv_hbm.at[pl.ds(wid * words, words)])


def _make_sc_edge(num_nodes, n_per_batch):
    npt = num_nodes // NTILES
    words = npt * DIM
    nlog = n_per_batch.bit_length() - 1
    assert (1 << nlog) == n_per_batch
    mesh = plsc.VectorSubcoreMesh(core_axis_name="c", subcore_axis_name="s")
    return pl.kernel(
        functools.partial(_sc_edge_body, npt, nlog),
        out_type=[jax.ShapeDtypeStruct((num_nodes * DIM,), jnp.float32),
                  jax.ShapeDtypeStruct((num_nodes * DIM,), jnp.float32)],
        mesh=mesh,
        scratch_types=[
            pltpu.VMEM((48,), jnp.int32),
            pltpu.VMEM((CHUNK, DIM), jnp.int32),
            pltpu.VMEM((CHUNK, DIM), jnp.int32),
            pltpu.VMEM((CHUNK,), jnp.float32),
            pltpu.VMEM((CHUNK,), jnp.float32),
            pltpu.VMEM((CHUNK,), jnp.int32),
            pltpu.VMEM((CHUNK,), jnp.int32),
            pltpu.VMEM((CHUNK,), jnp.int32),
            pltpu.VMEM((CHUNK,), jnp.int32),
            pltpu.VMEM((CHUNK,), jnp.int32),
            pltpu.VMEM((CHUNK,), jnp.int32),
            pltpu.VMEM((words,), jnp.float32),
            pltpu.VMEM((words,), jnp.float32),
            pltpu.SemaphoreType.DMA,
            pltpu.SemaphoreType.DMA,
            pltpu.SemaphoreType.DMA,
            pltpu.SemaphoreType.DMA,
        ],
        name="sc_edge_agg",
    )


def _row_spec(nrow, ncol):
    return pl.BlockSpec((nrow, ncol), lambda i: (i, 0))


def _full_spec(shape):
    return pl.BlockSpec(shape, lambda i: tuple(0 for _ in shape))


def _proj(x, v, qwT, qb, kwT, kb, swT, sb, vvwT, vvb, q_ref, k_ref, sp_ref):
    q_ref[...] = jnp.dot(x, qwT, preferred_element_type=jnp.float32) + qb
    k_ref[...] = jnp.dot(x, kwT, preferred_element_type=jnp.float32) + kb
    s = jnp.dot(x, swT, preferred_element_type=jnp.float32) + sb
    vv = jnp.dot(x, vvwT, preferred_element_type=jnp.float32) + vvb
    sp = jnp.concatenate([s, vv * v], axis=1)
    rows = sp.shape[0]
    spi = sp.reshape(rows, 2 * DIM // 32, 2, 16).transpose(0, 1, 3, 2)
    sp_ref[...] = spi.reshape(rows, 2 * DIM).astype(jnp.bfloat16)


def _tc_init_body(tok_ref, coord_ref, embed_ref, vwT_ref, vecb_ref,
                  qwT_ref, qb_ref, kwT_ref, kb_ref, swT_ref, sb_ref,
                  vvwT_ref, vvb_ref,
                  x_ref, v_ref, q_ref, k_ref, sp_ref):
    tok = tok_ref[...]
    oh = (tok == lax.broadcasted_iota(jnp.int32, (1, DIM), 1)
          ).astype(jnp.float32)
    x = jnp.dot(oh, embed_ref[...], preferred_element_type=jnp.float32)
    c = coord_ref[...]
    vwT = vwT_ref[...]
    v = (vecb_ref[...] + c[:, 0:1] * vwT[0:1, :]
         + c[:, 1:2] * vwT[1:2, :] + c[:, 2:3] * vwT[2:3, :])
    x_ref[...], v_ref[...] = x, v
    _proj(x, v, qwT_ref[...], qb_ref[...], kwT_ref[...], kb_ref[...],
          swT_ref[...], sb_ref[...], vvwT_ref[...], vvb_ref[...],
          q_ref, k_ref, sp_ref)


def _tc_mid_body(x_ref, v_ref, aggs_ref, aggv_ref, vmwT_ref, vmb_ref,
                 qwT_ref, qb_ref, kwT_ref, kb_ref, swT_ref, sb_ref,
                 vvwT_ref, vvb_ref,
                 x2_ref, v2_ref, q_ref, k_ref, sp_ref):
    x2 = x_ref[...] + aggs_ref[...]
    v2 = (v_ref[...]
          + jnp.dot(aggv_ref[...], vmwT_ref[...],
                    preferred_element_type=jnp.float32) + vmb_ref[...])
    x2_ref[...], v2_ref[...] = x2, v2
    _proj(x2, v2, qwT_ref[...], qb_ref[...], kwT_ref[...], kb_ref[...],
          swT_ref[...], sb_ref[...], vvwT_ref[...], vvb_ref[...],
          q_ref, k_ref, sp_ref)


def _tc_fin_body(x_ref, aggs_ref, x2_ref):
    x2_ref[...] = x_ref[...] + aggs_ref[...]


def _tc_w_body(q_ref, k_ref, w_ref):
    res = lax.dot_general(
        q_ref[...], k_ref[...], (((1,), (1,)), ((), ())),
        preferred_element_type=jnp.float32)
    w_ref[...] = res.reshape(w_ref.shape)


def _tc_init(nb, tok, coords, embed, vwT, vecb, lw):
    grid = nb // BLK
    nd = jax.ShapeDtypeStruct((nb, DIM), jnp.float32)
    sp = jax.ShapeDtypeStruct((nb, 2 * DIM), jnp.bfloat16)
    return pl.pallas_call(
        _tc_init_body,
        grid=(grid,),
        in_specs=[_row_spec(BLK, 1), _row_spec(BLK, 3),
                  _full_spec((DIM, DIM)), _full_spec((3, DIM)),
                  _full_spec((1, DIM)),
                  _full_spec((DIM, DIM)), _full_spec((1, DIM)),
                  _full_spec((DIM, DIM)), _full_spec((1, DIM)),
                  _full_spec((DIM, DIM)), _full_spec((1, DIM)),
                  _full_spec((DIM, DIM)), _full_spec((1, DIM))],
        out_specs=[_row_spec(BLK, DIM)] * 4 + [_row_spec(BLK, 2 * DIM)],
        out_shape=[nd, nd, nd, nd, sp],
        name="tc_init_proj",
    )(tok, coords, embed, vwT, vecb, *lw)


def _tc_mid(nb, x, v, aggs, aggv, vmwT, vmb, lw):
    grid = nb // BLK
    nd = jax.ShapeDtypeStruct((nb, DIM), jnp.float32)
    sp = jax.ShapeDtypeStruct((nb, 2 * DIM), jnp.bfloat16)
    return pl.pallas_call(
        _tc_mid_body,
        grid=(grid,),
        in_specs=[_row_spec(BLK, DIM)] * 4
        + [_full_spec((DIM, DIM)), _full_spec((1, DIM)),
           _full_spec((DIM, DIM)), _full_spec((1, DIM)),
           _full_spec((DIM, DIM)), _full_spec((1, DIM)),
           _full_spec((DIM, DIM)), _full_spec((1, DIM)),
           _full_spec((DIM, DIM)), _full_spec((1, DIM))],
        out_specs=[_row_spec(BLK, DIM)] * 4 + [_row_spec(BLK, 2 * DIM)],
        out_shape=[nd, nd, nd, nd, sp],
        name="tc_mid_proj",
    )(x, v, aggs, aggv, vmwT, vmb, *lw)


def _tc_w(nb, n, q, k):
    nbat = nb // n
    grid_r = n // BLK
    return pl.pallas_call(
        _tc_w_body,
        grid=(nbat, grid_r),
        in_specs=[pl.BlockSpec((BLK, DIM), lambda b, r: (b * grid_r + r, 0)),
                  pl.BlockSpec((n, DIM), lambda b, r: (b, 0))],
        out_specs=pl.BlockSpec((BLK * n // DIM, DIM),
                               lambda b, r: (b * grid_r + r, 0)),
        out_shape=jax.ShapeDtypeStruct((nb * n // DIM, DIM), jnp.float32),
        name="tc_qkT",
    )(q, k)


def _tc_fin(nb, x, aggs):
    grid = nb // BLK
    return pl.pallas_call(
        _tc_fin_body,
        grid=(grid,),
        in_specs=[_row_spec(BLK, DIM)] * 2,
        out_specs=_row_spec(BLK, DIM),
        out_shape=jax.ShapeDtypeStruct((nb, DIM), jnp.float32),
        name="tc_fin",
    )(x, aggs)


def kernel(src_tokens, padded_coordinates, src_distance, src_edge_type,
           edge_index, params):
    b, n = src_tokens.shape
    nb = b * n
    num_layers = len(params['layers'])

    tok = src_tokens.reshape(nb, 1).astype(jnp.int32)
    coords = padded_coordinates.reshape(nb, 3).astype(jnp.float32)
    rows = edge_index[0].astype(jnp.int32)
    cols = edge_index[1].astype(jnp.int32)
    e = rows.shape[0]
    epad = ((e + 4 * CHUNK + 7) // 8) * 8 + 8
    rows_p = jnp.concatenate(
        [rows, jnp.full((epad - e,), nb - 1, jnp.int32)])
    cols_p = jnp.concatenate([cols, jnp.zeros((epad - e,), jnp.int32)])
    bounds = (jnp.arange(NTILES + 1, dtype=jnp.int32) * (nb // NTILES))
    off = jnp.searchsorted(rows, bounds, side='left').astype(jnp.int32)
    off48 = jnp.concatenate([off, jnp.full((15,), e, jnp.int32)])

    embed = params['embed']
    vwT = params['vec_w'].T
    vecb = params['vec_b'].reshape(1, DIM)

    def layer_weights(lp):
        kvw = lp['kv_w']
        kvb = lp['kv_b']
        return (lp['q_w'].T, lp['q_b'].reshape(1, DIM),
                kvw[0:DIM].T, kvb[0:DIM].reshape(1, DIM),
                kvw[DIM:2 * DIM].T, kvb[DIM:2 * DIM].reshape(1, DIM),
                kvw[2 * DIM:].T, kvb[2 * DIM:].reshape(1, DIM))

    lws = [layer_weights(lp) for lp in params['layers']]
    sc_edge = _make_sc_edge(nb, n)

    x, v, q, k, sp = _tc_init(nb, tok, coords, embed, vwT, vecb, lws[0])
    for l in range(num_layers):
        w = _tc_w(nb, n, q, k)
        sp_i = lax.bitcast_convert_type(sp.reshape(nb, DIM, 2), jnp.int32)
        aggs_f, aggv_f = sc_edge(w.reshape(nb * n), sp_i, rows_p, cols_p,
                                 off48)
        aggs = aggs_f.reshape(nb, DIM)
        aggv = aggv_f.reshape(nb, DIM)
        if l + 1 < num_layers:
            lp = params['layers'][l]
            x, v, q, k, sp = _tc_mid(
                nb, x, v, aggs, aggv, lp['vm_w'].T,
                lp['vm_b'].reshape(1, DIM), lws[l + 1])
        else:
            x = _tc_fin(nb, x, aggs)

    encoder_rep = x.reshape(b, n, DIM)
    padding_mask = src_tokens == 0
    return encoder_rep, padding_mask


# speedup vs baseline: 1.5920x; 1.5920x over previous
import functools

import jax
import jax.numpy as jnp
from jax import lax
from jax.experimental import pallas as pl
from jax.experimental.pallas import tpu as pltpu
from jax.experimental.pallas import tpu_sc as plsc

DIM = 128
NSLICE = DIM // 16
CHUNK = 32
NTILES = 32
BLK = 512


def _sc_edge_body(npt, nlog,
                  w_hbm, sp_hbm, rows_hbm, cols_hbm, off_hbm,
                  aggs_hbm, aggv_hbm,
                  off_v, sp0, sp1, w0, w1, wi0, wi1,
                  rows0, rows1, cols0, cols1,
                  accs, accv, semg0, semg1, semi0, semi1):
    wid = lax.axis_index("s") * 2 + lax.axis_index("c")
    base_node = wid * npt
    words = npt * DIM
    nmask = (1 << nlog) - 1

    pltpu.sync_copy(off_hbm, off_v)
    lo = off_v[pl.ds(wid, 16)][0]
    hi = off_v[pl.ds(wid + 1, 16)][0]
    start = jnp.bitwise_and(lo, jnp.int32(~7))
    nch = (hi - start + (CHUNK - 1)) // CHUNK
    npairs = (nch + 1) // 2

    zeros16 = jnp.zeros((16,), jnp.float32)

    def zbody(t, c):
        for u in range(4):
            accs[pl.ds(t * 64 + 16 * u, 16)] = zeros16
            accv[pl.ds(t * 64 + 16 * u, 16)] = zeros16
        return c

    lax.fori_loop(0, words // 64, zbody, 0)

    def ds8(x):
        return pl.ds(pl.multiple_of(x, 8), CHUNK)

    def drain(dst, sem):
        pltpu.make_async_copy(rows_hbm.at[pl.ds(0, CHUNK)], dst, sem).wait()

    def drain_sp(dst, sem):
        pltpu.make_async_copy(sp_hbm.at[pl.ds(0, CHUNK)], dst, sem).wait()

    def drain_w(dst, sem):
        pltpu.make_async_copy(w_hbm.at[pl.ds(0, CHUNK)], dst, sem).wait()

    def fire_gathers(rowsr, colsr, wir, wr, spr, sem):
        for h in range(CHUNK // 16):
            r16 = rowsr[pl.ds(16 * h, 16)]
            c16 = colsr[pl.ds(16 * h, 16)]
            widx = ((jnp.bitwise_and(r16, nmask) << nlog)
                    + jnp.bitwise_and(c16, nmask)
                    + ((r16 >> nlog) << (2 * nlog)))
            wir[pl.ds(16 * h, 16)] = widx
        pltpu.async_copy(w_hbm.at[wir], wr, sem)
        pltpu.async_copy(sp_hbm.at[colsr], spr, sem)

    lane = lax.broadcasted_iota(jnp.int32, (16,), 0)

    def compute(e0c, rvecs, wr, spr, carry):
        prev_rb, cs, cp = carry
        for h in range(CHUNK // 16):
            r16 = rvecs[h]
            w16 = wr[pl.ds(16 * h, 16)]
            eid16 = (e0c + 16 * h) + lane
            validm = jnp.where(
                jnp.logical_and(eid16 >= lo, eid16 < hi),
                jnp.float32(1.0), jnp.float32(0.0))
            wm = w16 * validm
            rloc = jnp.clip(r16 - base_node, 0, npt - 1) * DIM
            for e in range(16):
                ei = 16 * h + e
                rb_e = rloc[e]
                changed = rb_e != prev_rb

                @pl.when(changed)
                def _flush(prev_rb=prev_rb, cs=cs, cp=cp):
                    for j in range(NSLICE):
                        accs[pl.ds(prev_rb + 16 * j, 16)] = cs[j]
                        accv[pl.ds(prev_rb + 16 * j, 16)] = cp[j]

                keep = jnp.where(changed, jnp.float32(0.0),
                                 jnp.float32(1.0))
                sel = jnp.full((16,), e, jnp.int32)
                w_b = wm.at[sel].get(mode="promise_in_bounds")
                cs = tuple(
                    cs[j] * keep + w_b * spr[ei, pl.ds(16 * j, 16)]
                    for j in range(NSLICE))
                cp = tuple(
                    cp[j] * keep + w_b * spr[ei, pl.ds(DIM + 16 * j, 16)]
                    for j in range(NSLICE))
                prev_rb = rb_e
        return prev_rb, cs, cp

    pltpu.sync_copy(rows_hbm.at[ds8(start)], rows0)
    pltpu.sync_copy(cols_hbm.at[ds8(start)], cols0)
    fire_gathers(rows0, cols0, wi0, w0, sp0, semg0)
    pltpu.async_copy(rows_hbm.at[ds8(start + CHUNK)], rows1, semi1)
    pltpu.async_copy(cols_hbm.at[ds8(start + CHUNK)], cols1, semi1)

    def pbody(gg, carry):
        ca = start + (2 * gg) * CHUNK
        drain_w(w0, semg0)
        drain_sp(sp0, semg0)
        ra = [rows0[pl.ds(16 * h, 16)] for h in range(CHUNK // 16)]
        drain(rows1, semi1)
        drain(cols1, semi1)
        fire_gathers(rows1, cols1, wi1, w1, sp1, semg1)
        pltpu.async_copy(rows_hbm.at[ds8(ca + 2 * CHUNK)], rows0, semi0)
        pltpu.async_copy(cols_hbm.at[ds8(ca + 2 * CHUNK)], cols0, semi0)
        carry = compute(ca, ra, w0, sp0, carry)
        drain_w(w1, semg1)
        drain_sp(sp1, semg1)
        rb = [rows1[pl.ds(16 * h, 16)] for h in range(CHUNK // 16)]
        drain(rows0, semi0)
        drain(cols0, semi0)
        fire_gathers(rows0, cols0, wi0, w0, sp0, semg0)
        pltpu.async_copy(rows_hbm.at[ds8(ca + 3 * CHUNK)], rows1, semi1)
        pltpu.async_copy(cols_hbm.at[ds8(ca + 3 * CHUNK)], cols1, semi1)
        carry = compute(ca + CHUNK, rb, w1, sp1, carry)
        return carry

    zero8 = tuple(jnp.zeros((16,), jnp.float32) for _ in range(NSLICE))
    carry0 = (jnp.int32(0), zero8, zero8)
    prev_rb, cs, cp = lax.fori_loop(0, npairs, pbody, carry0)

    for j in range(NSLICE):
        accs[pl.ds(prev_rb + 16 * j, 16)] = cs[j]
        accv[pl.ds(prev_rb + 16 * j, 16)] = cp[j]

    drain_w(w0, semg0)
    drain_sp(sp0, semg0)
    drain(rows1, semi1)
    drain(cols1, semi1)

    pltpu.sync_copy(accs, aggs_hbm.at[pl.ds(wid * words, words)])
    pltpu.sync_copy(accv, aggv_hbm.at[pl.ds(wid * words, words)])


def _make_sc_edge(num_nodes, n_per_batch):
    npt = num_nodes // NTILES
    words = npt * DIM
    nlog = n_per_batch.bit_length() - 1
    assert (1 << nlog) == n_per_batch
    mesh = plsc.VectorSubcoreMesh(core_axis_name="c", subcore_axis_name="s")
    return pl.kernel(
        functools.partial(_sc_edge_body, npt, nlog),
        out_type=[jax.ShapeDtypeStruct((num_nodes * DIM,), jnp.float32),
                  jax.ShapeDtypeStruct((num_nodes * DIM,), jnp.float32)],
        mesh=mesh,
        scratch_types=[
            pltpu.VMEM((48,), jnp.int32),
            pltpu.VMEM((CHUNK, 2 * DIM), jnp.float32),
            pltpu.VMEM((CHUNK, 2 * DIM), jnp.float32),
            pltpu.VMEM((CHUNK,), jnp.float32),
            pltpu.VMEM((CHUNK,), jnp.float32),
            pltpu.VMEM((CHUNK,), jnp.int32),
            pltpu.VMEM((CHUNK,), jnp.int32),
            pltpu.VMEM((CHUNK,), jnp.int32),
            pltpu.VMEM((CHUNK,), jnp.int32),
            pltpu.VMEM((CHUNK,), jnp.int32),
            pltpu.VMEM((CHUNK,), jnp.int32),
            pltpu.VMEM((words,), jnp.float32),
            pltpu.VMEM((words,), jnp.float32),
            pltpu.SemaphoreType.DMA,
            pltpu.SemaphoreType.DMA,
            pltpu.SemaphoreType.DMA,
            pltpu.SemaphoreType.DMA,
        ],
        name="sc_edge_agg",
    )


def _row_spec(nrow, ncol):
    return pl.BlockSpec((nrow, ncol), lambda i: (i, 0))


def _full_spec(shape):
    return pl.BlockSpec(shape, lambda i: tuple(0 for _ in shape))


def _proj(x, v, qwT, qb, kwT, kb, swT, sb, vvwT, vvb, q_ref, k_ref, sp_ref):
    q_ref[...] = jnp.dot(x, qwT, preferred_element_type=jnp.float32) + qb
    k_ref[...] = jnp.dot(x, kwT, preferred_element_type=jnp.float32) + kb
    sp_ref[:, 0:DIM] = (
        jnp.dot(x, swT, preferred_element_type=jnp.float32) + sb)
    vv = jnp.dot(x, vvwT, preferred_element_type=jnp.float32) + vvb
    sp_ref[:, DIM:2 * DIM] = vv * v


def _tc_init_body(tok_ref, coord_ref, embed_ref, vwT_ref, vecb_ref,
                  qwT_ref, qb_ref, kwT_ref, kb_ref, swT_ref, sb_ref,
                  vvwT_ref, vvb_ref,
                  x_ref, v_ref, q_ref, k_ref, sp_ref):
    tok = tok_ref[...]
    oh = (tok == lax.broadcasted_iota(jnp.int32, (1, DIM), 1)
          ).astype(jnp.float32)
    x = jnp.dot(oh, embed_ref[...], preferred_element_type=jnp.float32)
    c = coord_ref[...]
    vwT = vwT_ref[...]
    v = (vecb_ref[...] + c[:, 0:1] * vwT[0:1, :]
         + c[:, 1:2] * vwT[1:2, :] + c[:, 2:3] * vwT[2:3, :])
    x_ref[...], v_ref[...] = x, v
    _proj(x, v, qwT_ref[...], qb_ref[...], kwT_ref[...], kb_ref[...],
          swT_ref[...], sb_ref[...], vvwT_ref[...], vvb_ref[...],
          q_ref, k_ref, sp_ref)


def _tc_mid_body(x_ref, v_ref, aggs_ref, aggv_ref, vmwT_ref, vmb_ref,
                 qwT_ref, qb_ref, kwT_ref, kb_ref, swT_ref, sb_ref,
                 vvwT_ref, vvb_ref,
                 x2_ref, v2_ref, q_ref, k_ref, sp_ref):
    x2 = x_ref[...] + aggs_ref[...]
    v2 = (v_ref[...]
          + jnp.dot(aggv_ref[...], vmwT_ref[...],
                    preferred_element_type=jnp.float32) + vmb_ref[...])
    x2_ref[...], v2_ref[...] = x2, v2
    _proj(x2, v2, qwT_ref[...], qb_ref[...], kwT_ref[...], kb_ref[...],
          swT_ref[...], sb_ref[...], vvwT_ref[...], vvb_ref[...],
          q_ref, k_ref, sp_ref)


def _tc_fin_body(x_ref, aggs_ref, x2_ref):
    x2_ref[...] = x_ref[...] + aggs_ref[...]


def _tc_w_body(q_ref, k_ref, w_ref):
    res = lax.dot_general(
        q_ref[...], k_ref[...], (((1,), (1,)), ((), ())),
        preferred_element_type=jnp.float32)
    w_ref[...] = res.reshape(w_ref.shape)


def _tc_init(nb, tok, coords, embed, vwT, vecb, lw):
    grid = nb // BLK
    nd = jax.ShapeDtypeStruct((nb, DIM), jnp.float32)
    sp = jax.ShapeDtypeStruct((nb, 2 * DIM), jnp.float32)
    return pl.pallas_call(
        _tc_init_body,
        grid=(grid,),
        in_specs=[_row_spec(BLK, 1), _row_spec(BLK, 3),
                  _full_spec((DIM, DIM)), _full_spec((3, DIM)),
                  _full_spec((1, DIM)),
                  _full_spec((DIM, DIM)), _full_spec((1, DIM)),
                  _full_spec((DIM, DIM)), _full_spec((1, DIM)),
                  _full_spec((DIM, DIM)), _full_spec((1, DIM)),
                  _full_spec((DIM, DIM)), _full_spec((1, DIM))],
        out_specs=[_row_spec(BLK, DIM)] * 4 + [_row_spec(BLK, 2 * DIM)],
        out_shape=[nd, nd, nd, nd, sp],
        name="tc_init_proj",
    )(tok, coords, embed, vwT, vecb, *lw)


def _tc_mid(nb, x, v, aggs, aggv, vmwT, vmb, lw):
    grid = nb // BLK
    nd = jax.ShapeDtypeStruct((nb, DIM), jnp.float32)
    sp = jax.ShapeDtypeStruct((nb, 2 * DIM), jnp.float32)
    return pl.pallas_call(
        _tc_mid_body,
        grid=(grid,),
        in_specs=[_row_spec(BLK, DIM)] * 4
        + [_full_spec((DIM, DIM)), _full_spec((1, DIM)),
           _full_spec((DIM, DIM)), _full_spec((1, DIM)),
           _full_spec((DIM, DIM)), _full_spec((1, DIM)),
           _full_spec((DIM, DIM)), _full_spec((1, DIM)),
           _full_spec((DIM, DIM)), _full_spec((1, DIM))],
        out_specs=[_row_spec(BLK, DIM)] * 4 + [_row_spec(BLK, 2 * DIM)],
        out_shape=[nd, nd, nd, nd, sp],
        name="tc_mid_proj",
    )(x, v, aggs, aggv, vmwT, vmb, *lw)


def _tc_w(nb, n, q, k):
    nbat = nb // n
    grid_r = n // BLK
    return pl.pallas_call(
        _tc_w_body,
        grid=(nbat, grid_r),
        in_specs=[pl.BlockSpec((BLK, DIM), lambda b, r: (b * grid_r + r, 0)),
                  pl.BlockSpec((n, DIM), lambda b, r: (b, 0))],
        out_specs=pl.BlockSpec((BLK * n // DIM, DIM),
                               lambda b, r: (b * grid_r + r, 0)),
        out_shape=jax.ShapeDtypeStruct((nb * n // DIM, DIM), jnp.float32),
        name="tc_qkT",
    )(q, k)


def _tc_fin(nb, x, aggs):
    grid = nb // BLK
    return pl.pallas_call(
        _tc_fin_body,
        grid=(grid,),
        in_specs=[_row_spec(BLK, DIM)] * 2,
        out_specs=_row_spec(BLK, DIM),
        out_shape=jax.ShapeDtypeStruct((nb, DIM), jnp.float32),
        name="tc_fin",
    )(x, aggs)


def kernel(src_tokens, padded_coordinates, src_distance, src_edge_type,
           edge_index, params):
    b, n = src_tokens.shape
    nb = b * n
    num_layers = len(params['layers'])

    tok = src_tokens.reshape(nb, 1).astype(jnp.int32)
    coords = padded_coordinates.reshape(nb, 3).astype(jnp.float32)
    rows = edge_index[0].astype(jnp.int32)
    cols = edge_index[1].astype(jnp.int32)
    e = rows.shape[0]
    epad = ((e + 4 * CHUNK + 7) // 8) * 8 + 8
    rows_p = jnp.concatenate(
        [rows, jnp.full((epad - e,), nb - 1, jnp.int32)])
    cols_p = jnp.concatenate([cols, jnp.zeros((epad - e,), jnp.int32)])
    bounds = (jnp.arange(NTILES + 1, dtype=jnp.int32) * (nb // NTILES))
    off = jnp.searchsorted(rows, bounds, side='left').astype(jnp.int32)
    off48 = jnp.concatenate([off, jnp.full((15,), e, jnp.int32)])

    embed = params['embed']
    vwT = params['vec_w'].T
    vecb = params['vec_b'].reshape(1, DIM)

    def layer_weights(lp):
        kvw = lp['kv_w']
        kvb = lp['kv_b']
        return (lp['q_w'].T, lp['q_b'].reshape(1, DIM),
                kvw[0:DIM].T, kvb[0:DIM].reshape(1, DIM),
                kvw[DIM:2 * DIM].T, kvb[DIM:2 * DIM].reshape(1, DIM),
                kvw[2 * DIM:].T, kvb[2 * DIM:].reshape(1, DIM))

    lws = [layer_weights(lp) for lp in params['layers']]
    sc_edge = _make_sc_edge(nb, n)

    x, v, q, k, sp = _tc_init(nb, tok, coords, embed, vwT, vecb, lws[0])
    for l in range(num_layers):
        w = _tc_w(nb, n, q, k)
        aggs_f, aggv_f = sc_edge(w.reshape(nb * n), sp, rows_p, cols_p,
                                 off48)
        aggs = aggs_f.reshape(nb, DIM)
        aggv = aggv_f.reshape(nb, DIM)
        if l + 1 < num_layers:
            lp = params['layers'][l]
            x, v, q, k, sp = _tc_mid(
                nb, x, v, aggs, aggv, lp['vm_w'].T,
                lp['vm_b'].reshape(1, DIM), lws[l + 1])
        else:
            x = _tc_fin(nb, x, aggs)

    encoder_rep = x.reshape(b, n, DIM)
    padding_mask = src_tokens == 0
    return encoder_rep, padding_mask


# speedup vs baseline: 1.6650x; 1.0459x over previous
import functools

import jax
import jax.numpy as jnp
from jax import lax
from jax.experimental import pallas as pl
from jax.experimental.pallas import tpu as pltpu
from jax.experimental.pallas import tpu_sc as plsc

DIM = 128
NSLICE = DIM // 16
CHUNK = 32
NTILES = 32
BLK = 512


def _sc_edge_body(npt, nlog,
                  w_hbm, sp_hbm, rows_hbm, cols_hbm, off_hbm,
                  aggs_hbm, aggv_hbm,
                  off_v, sp0, sp1, w0, w1, wi0, wi1,
                  rows0, rows1, cols0, cols1,
                  accs, accv, semg0, semg1, semi0, semi1):
    wid = lax.axis_index("s") * 2 + lax.axis_index("c")
    base_node = wid * npt
    words = npt * DIM
    nmask = (1 << nlog) - 1

    pltpu.sync_copy(off_hbm, off_v)
    lo = off_v[pl.ds(wid, 16)][0]
    hi = off_v[pl.ds(wid + 1, 16)][0]
    start = jnp.bitwise_and(lo, jnp.int32(~7))
    nch = (hi - start + (CHUNK - 1)) // CHUNK
    npairs = (nch + 1) // 2

    zeros16 = jnp.zeros((16,), jnp.float32)

    def zbody(t, c):
        for u in range(4):
            accs[pl.ds(t * 64 + 16 * u, 16)] = zeros16
            accv[pl.ds(t * 64 + 16 * u, 16)] = zeros16
        return c

    lax.fori_loop(0, words // 64, zbody, 0)

    def ds8(x):
        return pl.ds(pl.multiple_of(x, 8), CHUNK)

    def drain(dst, sem):
        pltpu.make_async_copy(rows_hbm.at[pl.ds(0, CHUNK)], dst, sem).wait()

    def drain_sp(dst, sem):
        pltpu.make_async_copy(sp_hbm.at[pl.ds(0, CHUNK)], dst, sem).wait()

    def drain_w(dst, sem):
        pltpu.make_async_copy(w_hbm.at[pl.ds(0, CHUNK)], dst, sem).wait()

    def fire_gathers(rowsr, colsr, wir, wr, spr, sem):
        for h in range(CHUNK // 16):
            r16 = rowsr[pl.ds(16 * h, 16)]
            c16 = colsr[pl.ds(16 * h, 16)]
            widx = ((jnp.bitwise_and(r16, nmask) << nlog)
                    + jnp.bitwise_and(c16, nmask)
                    + ((r16 >> nlog) << (2 * nlog)))
            wir[pl.ds(16 * h, 16)] = widx
        pltpu.async_copy(w_hbm.at[wir], wr, sem)
        pltpu.async_copy(sp_hbm.at[colsr], spr, sem)

    lane = lax.broadcasted_iota(jnp.int32, (16,), 0)

    def compute(e0c, rvecs, wr, spr, carry):
        prev_rb, cs, cp = carry
        for h in range(CHUNK // 16):
            r16 = rvecs[h]
            w16 = wr[pl.ds(16 * h, 16)]
            eid16 = (e0c + 16 * h) + lane
            validm = jnp.where(
                jnp.logical_and(eid16 >= lo, eid16 < hi),
                jnp.float32(1.0), jnp.float32(0.0))
            wm = w16 * validm
            rloc = jnp.clip(r16 - base_node, 0, npt - 1) * DIM
            for e in range(16):
                ei = 16 * h + e
                rb_e = rloc[e]
                changed = rb_e != prev_rb

                @pl.when(changed)
                def _flush(prev_rb=prev_rb, cs=cs, cp=cp):
                    for j in range(NSLICE):
                        accs[pl.ds(prev_rb + 16 * j, 16)] = cs[j]
                        accv[pl.ds(prev_rb + 16 * j, 16)] = cp[j]

                keep = jnp.where(changed, jnp.float32(0.0),
                                 jnp.float32(1.0))
                sel = jnp.full((16,), e, jnp.int32)
                w_b = wm.at[sel].get(mode="promise_in_bounds")
                ncs, ncp = list(cs), list(cp)
                for u in range(2 * DIM // 32):
                    xi = spr[ei, pl.ds(16 * u, 16)]
                    vlo = lax.bitcast_convert_type(xi << 16, jnp.float32)
                    vhi = lax.bitcast_convert_type(
                        jnp.bitwise_and(xi, jnp.int32(-65536)),
                        jnp.float32)
                    if u < DIM // 32:
                        ncs[2 * u] = cs[2 * u] * keep + w_b * vlo
                        ncs[2 * u + 1] = cs[2 * u + 1] * keep + w_b * vhi
                    else:
                        t = u - DIM // 32
                        ncp[2 * t] = cp[2 * t] * keep + w_b * vlo
                        ncp[2 * t + 1] = cp[2 * t + 1] * keep + w_b * vhi
                cs, cp = tuple(ncs), tuple(ncp)
                prev_rb = rb_e
        return prev_rb, cs, cp

    pltpu.sync_copy(rows_hbm.at[ds8(start)], rows0)
    pltpu.sync_copy(cols_hbm.at[ds8(start)], cols0)
    fire_gathers(rows0, cols0, wi0, w0, sp0, semg0)
    pltpu.async_copy(rows_hbm.at[ds8(start + CHUNK)], rows1, semi1)
    pltpu.async_copy(cols_hbm.at[ds8(start + CHUNK)], cols1, semi1)

    def pbody(gg, carry):
        ca = start + (2 * gg) * CHUNK
        drain_w(w0, semg0)
        drain_sp(sp0, semg0)
        ra = [rows0[pl.ds(16 * h, 16)] for h in range(CHUNK // 16)]
        drain(rows1, semi1)
        drain(cols1, semi1)
        fire_gathers(rows1, cols1, wi1, w1, sp1, semg1)
        pltpu.async_copy(rows_hbm.at[ds8(ca + 2 * CHUNK)], rows0, semi0)
        pltpu.async_copy(cols_hbm.at[ds8(ca + 2 * CHUNK)], cols0, semi0)
        carry = compute(ca, ra, w0, sp0, carry)
        drain_w(w1, semg1)
        drain_sp(sp1, semg1)
        rb = [rows1[pl.ds(16 * h, 16)] for h in range(CHUNK // 16)]
        drain(rows0, semi0)
        drain(cols0, semi0)
        fire_gathers(rows0, cols0, wi0, w0, sp0, semg0)
        pltpu.async_copy(rows_hbm.at[ds8(ca + 3 * CHUNK)], rows1, semi1)
        pltpu.async_copy(cols_hbm.at[ds8(ca + 3 * CHUNK)], cols1, semi1)
        carry = compute(ca + CHUNK, rb, w1, sp1, carry)
        return carry

    zero8 = tuple(jnp.zeros((16,), jnp.float32) for _ in range(NSLICE))
    carry0 = (jnp.int32(0), zero8, zero8)
    prev_rb, cs, cp = lax.fori_loop(0, npairs, pbody, carry0)

    for j in range(NSLICE):
        accs[pl.ds(prev_rb + 16 * j, 16)] = cs[j]
        accv[pl.ds(prev_rb + 16 * j, 16)] = cp[j]

    drain_w(w0, semg0)
    drain_sp(sp0, semg0)
    drain(rows1, semi1)
    drain(cols1, semi1)

    pltpu.sync_copy(accs, aggs_hbm.at[pl.ds(wid * words, words)])
    pltpu.sync_copy(accv, aggv_hbm.at[pl.ds(wid * words, words)])


def _make_sc_edge(num_nodes, n_per_batch):
    npt = num_nodes // NTILES
    words = npt * DIM
    nlog = n_per_batch.bit_length() - 1
    assert (1 << nlog) == n_per_batch
    mesh = plsc.VectorSubcoreMesh(core_axis_name="c", subcore_axis_name="s")
    return pl.kernel(
        functools.partial(_sc_edge_body, npt, nlog),
        out_type=[jax.ShapeDtypeStruct((num_nodes * DIM,), jnp.float32),
                  jax.ShapeDtypeStruct((num_nodes * DIM,), jnp.float32)],
        mesh=mesh,
        scratch_types=[
            pltpu.VMEM((48,), jnp.int32),
            pltpu.VMEM((CHUNK, DIM), jnp.int32),
            pltpu.VMEM((CHUNK, DIM), jnp.int32),
            pltpu.VMEM((CHUNK,), jnp.float32),
            pltpu.VMEM((CHUNK,), jnp.float32),
            pltpu.VMEM((CHUNK,), jnp.int32),
            pltpu.VMEM((CHUNK,), jnp.int32),
            pltpu.VMEM((CHUNK,), jnp.int32),
            pltpu.VMEM((CHUNK,), jnp.int32),
            pltpu.VMEM((CHUNK,), jnp.int32),
            pltpu.VMEM((CHUNK,), jnp.int32),
            pltpu.VMEM((words,), jnp.float32),
            pltpu.VMEM((words,), jnp.float32),
            pltpu.SemaphoreType.DMA,
            pltpu.SemaphoreType.DMA,
            pltpu.SemaphoreType.DMA,
            pltpu.SemaphoreType.DMA,
        ],
        name="sc_edge_agg",
    )


def _row_spec(nrow, ncol):
    return pl.BlockSpec((nrow, ncol), lambda i: (i, 0))


def _full_spec(shape):
    return pl.BlockSpec(shape, lambda i: tuple(0 for _ in shape))


def _proj(x, v, qwT, qb, kwT, kb, swT, sb, vvwT, vvb, q_ref, k_ref, sp_ref):
    q_ref[...] = jnp.dot(x, qwT, preferred_element_type=jnp.float32) + qb
    k_ref[...] = jnp.dot(x, kwT, preferred_element_type=jnp.float32) + kb
    s = jnp.dot(x, swT, preferred_element_type=jnp.float32) + sb
    vv = jnp.dot(x, vvwT, preferred_element_type=jnp.float32) + vvb
    sp_ref[...] = jnp.concatenate([s, vv * v], axis=1).astype(jnp.bfloat16)


def _tc_init_body(tok_ref, coord_ref, embed_ref, vwT_ref, vecb_ref,
                  qwT_ref, qb_ref, kwT_ref, kb_ref, swT_ref, sb_ref,
                  vvwT_ref, vvb_ref,
                  x_ref, v_ref, q_ref, k_ref, sp_ref):
    tok = tok_ref[...]
    oh = (tok == lax.broadcasted_iota(jnp.int32, (1, DIM), 1)
          ).astype(jnp.float32)
    x = jnp.dot(oh, embed_ref[...], preferred_element_type=jnp.float32)
    c = coord_ref[...]
    vwT = vwT_ref[...]
    v = (vecb_ref[...] + c[:, 0:1] * vwT[0:1, :]
         + c[:, 1:2] * vwT[1:2, :] + c[:, 2:3] * vwT[2:3, :])
    x_ref[...], v_ref[...] = x, v
    _proj(x, v, qwT_ref[...], qb_ref[...], kwT_ref[...], kb_ref[...],
          swT_ref[...], sb_ref[...], vvwT_ref[...], vvb_ref[...],
          q_ref, k_ref, sp_ref)


def _tc_mid_body(x_ref, v_ref, aggs_ref, aggv_ref, vmwT_ref, vmb_ref,
                 qwT_ref, qb_ref, kwT_ref, kb_ref, swT_ref, sb_ref,
                 vvwT_ref, vvb_ref,
                 x2_ref, v2_ref, q_ref, k_ref, sp_ref):
    x2 = x_ref[...] + aggs_ref[...]
    v2 = (v_ref[...]
          + jnp.dot(aggv_ref[...], vmwT_ref[...],
                    preferred_element_type=jnp.float32) + vmb_ref[...])
    x2_ref[...], v2_ref[...] = x2, v2
    _proj(x2, v2, qwT_ref[...], qb_ref[...], kwT_ref[...], kb_ref[...],
          swT_ref[...], sb_ref[...], vvwT_ref[...], vvb_ref[...],
          q_ref, k_ref, sp_ref)


def _tc_fin_body(x_ref, aggs_ref, x2_ref):
    x2_ref[...] = x_ref[...] + aggs_ref[...]


def _tc_w_body(q_ref, k_ref, w_ref):
    res = lax.dot_general(
        q_ref[...], k_ref[...], (((1,), (1,)), ((), ())),
        preferred_element_type=jnp.float32)
    w_ref[...] = res.reshape(w_ref.shape)


def _tc_init(nb, tok, coords, embed, vwT, vecb, lw):
    grid = nb // BLK
    nd = jax.ShapeDtypeStruct((nb, DIM), jnp.float32)
    sp = jax.ShapeDtypeStruct((nb, 2 * DIM), jnp.bfloat16)
    return pl.pallas_call(
        _tc_init_body,
        grid=(grid,),
        in_specs=[_row_spec(BLK, 1), _row_spec(BLK, 3),
                  _full_spec((DIM, DIM)), _full_spec((3, DIM)),
                  _full_spec((1, DIM)),
                  _full_spec((DIM, DIM)), _full_spec((1, DIM)),
                  _full_spec((DIM, DIM)), _full_spec((1, DIM)),
                  _full_spec((DIM, DIM)), _full_spec((1, DIM)),
                  _full_spec((DIM, DIM)), _full_spec((1, DIM))],
        out_specs=[_row_spec(BLK, DIM)] * 4 + [_row_spec(BLK, 2 * DIM)],
        out_shape=[nd, nd, nd, nd, sp],
        name="tc_init_proj",
    )(tok, coords, embed, vwT, vecb, *lw)


def _tc_mid(nb, x, v, aggs, aggv, vmwT, vmb, lw):
    grid = nb // BLK
    nd = jax.ShapeDtypeStruct((nb, DIM), jnp.float32)
    sp = jax.ShapeDtypeStruct((nb, 2 * DIM), jnp.bfloat16)
    return pl.pallas_call(
        _tc_mid_body,
        grid=(grid,),
        in_specs=[_row_spec(BLK, DIM)] * 4
        + [_full_spec((DIM, DIM)), _full_spec((1, DIM)),
           _full_spec((DIM, DIM)), _full_spec((1, DIM)),
           _full_spec((DIM, DIM)), _full_spec((1, DIM)),
           _full_spec((DIM, DIM)), _full_spec((1, DIM)),
           _full_spec((DIM, DIM)), _full_spec((1, DIM))],
        out_specs=[_row_spec(BLK, DIM)] * 4 + [_row_spec(BLK, 2 * DIM)],
        out_shape=[nd, nd, nd, nd, sp],
        name="tc_mid_proj",
    )(x, v, aggs, aggv, vmwT, vmb, *lw)


def _tc_w(nb, n, q, k):
    nbat = nb // n
    grid_r = n // BLK
    return pl.pallas_call(
        _tc_w_body,
        grid=(nbat, grid_r),
        in_specs=[pl.BlockSpec((BLK, DIM), lambda b, r: (b * grid_r + r, 0)),
                  pl.BlockSpec((n, DIM), lambda b, r: (b, 0))],
        out_specs=pl.BlockSpec((BLK * n // DIM, DIM),
                               lambda b, r: (b * grid_r + r, 0)),
        out_shape=jax.ShapeDtypeStruct((nb * n // DIM, DIM), jnp.float32),
        name="tc_qkT",
    )(q, k)


def _tc_fin(nb, x, aggs):
    grid = nb // BLK
    return pl.pallas_call(
        _tc_fin_body,
        grid=(grid,),
        in_specs=[_row_spec(BLK, DIM)] * 2,
        out_specs=_row_spec(BLK, DIM),
        out_shape=jax.ShapeDtypeStruct((nb, DIM), jnp.float32),
        name="tc_fin",
    )(x, aggs)


def kernel(src_tokens, padded_coordinates, src_distance, src_edge_type,
           edge_index, params):
    b, n = src_tokens.shape
    nb = b * n
    num_layers = len(params['layers'])

    tok = src_tokens.reshape(nb, 1).astype(jnp.int32)
    coords = padded_coordinates.reshape(nb, 3).astype(jnp.float32)
    rows = edge_index[0].astype(jnp.int32)
    cols = edge_index[1].astype(jnp.int32)
    e = rows.shape[0]
    epad = ((e + 4 * CHUNK + 7) // 8) * 8 + 8
    rows_p = jnp.concatenate(
        [rows, jnp.full((epad - e,), nb - 1, jnp.int32)])
    cols_p = jnp.concatenate([cols, jnp.zeros((epad - e,), jnp.int32)])
    bounds = (jnp.arange(NTILES + 1, dtype=jnp.int32) * (nb // NTILES))
    off = jnp.searchsorted(rows, bounds, side='left').astype(jnp.int32)
    off48 = jnp.concatenate([off, jnp.full((15,), e, jnp.int32)])

    embed = params['embed']
    fidx = jnp.arange(DIM)
    sigma = 32 * (fidx // 32) + (fidx % 32) // 2 + 16 * (fidx % 2)
    vwT = params['vec_w'].T[:, sigma]
    vecb = params['vec_b'][sigma].reshape(1, DIM)

    def layer_weights(lp):
        kvw = lp['kv_w']
        kvb = lp['kv_b']
        return (lp['q_w'].T, lp['q_b'].reshape(1, DIM),
                kvw[0:DIM].T, kvb[0:DIM].reshape(1, DIM),
                kvw[DIM:2 * DIM].T[:, sigma],
                kvb[DIM:2 * DIM][sigma].reshape(1, DIM),
                kvw[2 * DIM:].T[:, sigma],
                kvb[2 * DIM:][sigma].reshape(1, DIM))

    lws = [layer_weights(lp) for lp in params['layers']]
    sc_edge = _make_sc_edge(nb, n)

    x, v, q, k, sp = _tc_init(nb, tok, coords, embed, vwT, vecb, lws[0])
    for l in range(num_layers):
        w = _tc_w(nb, n, q, k)
        sp_i = lax.bitcast_convert_type(
            sp.reshape(nb, DIM, 2), jnp.int32)
        aggs_f, aggv_f = sc_edge(w.reshape(nb * n), sp_i, rows_p, cols_p,
                                 off48)
        aggs = aggs_f.reshape(nb, DIM)
        aggv = aggv_f.reshape(nb, DIM)
        if l + 1 < num_layers:
            lp = params['layers'][l]
            x, v, q, k, sp = _tc_mid(
                nb, x, v, aggs, aggv, lp['vm_w'].T[:, sigma],
                lp['vm_b'][sigma].reshape(1, DIM), lws[l + 1])
        else:
            x = _tc_fin(nb, x, aggs)

    encoder_rep = x.reshape(b, n, DIM)
    padding_mask = src_tokens == 0
    return encoder_rep, padding_mask


# speedup vs baseline: 1.8568x; 1.1152x over previous
import functools

import jax
import jax.numpy as jnp
from jax import lax
from jax.experimental import pallas as pl
from jax.experimental.pallas import tpu as pltpu
from jax.experimental.pallas import tpu_sc as plsc

DIM = 128
NSLICE = DIM // 16
CHUNK = 32
NTILES = 32
BLK = 512


def _sc_edge_body(npt, nlog,
                  w_hbm, sp_hbm, rows_hbm, cols_hbm, off_hbm,
                  aggs_hbm, aggv_hbm,
                  off_v, sp0, sp1, w0, w1, wi0, wi1,
                  rows0, rows1, cols0, cols1,
                  accs, accv, semg0, semg1, semi0, semi1):
    wid = lax.axis_index("s") * 2 + lax.axis_index("c")
    base_node = wid * npt
    words = npt * DIM
    nmask = (1 << nlog) - 1

    pltpu.sync_copy(off_hbm, off_v)
    lo = off_v[pl.ds(wid, 16)][0]
    hi = off_v[pl.ds(wid + 1, 16)][0]
    start = jnp.bitwise_and(lo, jnp.int32(~7))
    nch = (hi - start + (CHUNK - 1)) // CHUNK
    npairs = (nch + 1) // 2

    zeros16 = jnp.zeros((16,), jnp.float32)

    def zbody(t, c):
        for u in range(4):
            accs[pl.ds(t * 64 + 16 * u, 16)] = zeros16
            accv[pl.ds(t * 64 + 16 * u, 16)] = zeros16
        return c

    lax.fori_loop(0, words // 64, zbody, 0)

    def ds8(x):
        return pl.ds(pl.multiple_of(x, 8), CHUNK)

    def drain(dst, sem):
        pltpu.make_async_copy(rows_hbm.at[pl.ds(0, CHUNK)], dst, sem).wait()

    def drain_sp(dst, sem):
        pltpu.make_async_copy(sp_hbm.at[pl.ds(0, CHUNK)], dst, sem).wait()

    def drain_w(dst, sem):
        pltpu.make_async_copy(w_hbm.at[pl.ds(0, CHUNK)], dst, sem).wait()

    def fire_gathers(rowsr, colsr, wir, wr, spr, sem):
        for h in range(CHUNK // 16):
            r16 = rowsr[pl.ds(16 * h, 16)]
            c16 = colsr[pl.ds(16 * h, 16)]
            widx = ((jnp.bitwise_and(r16, nmask) << nlog)
                    + jnp.bitwise_and(c16, nmask)
                    + ((r16 >> nlog) << (2 * nlog)))
            wir[pl.ds(16 * h, 16)] = widx
        pltpu.async_copy(w_hbm.at[wir], wr, sem)
        pltpu.async_copy(sp_hbm.at[colsr], spr, sem)

    lane = lax.broadcasted_iota(jnp.int32, (16,), 0)

    def compute(e0c, rvecs, wr, spr, carry):
        prev_rb, cs, cp = carry
        for h in range(CHUNK // 16):
            r16 = rvecs[h]
            w16 = wr[pl.ds(16 * h, 16)]
            eid16 = (e0c + 16 * h) + lane
            validm = jnp.where(
                jnp.logical_and(eid16 >= lo, eid16 < hi),
                jnp.float32(1.0), jnp.float32(0.0))
            wm = w16 * validm
            rloc = jnp.clip(r16 - base_node, 0, npt - 1) * DIM
            for e in range(16):
                ei = 16 * h + e
                rb_e = rloc[e]
                changed = rb_e != prev_rb

                @pl.when(changed)
                def _flush(prev_rb=prev_rb, cs=cs, cp=cp):
                    for j in range(NSLICE):
                        accs[pl.ds(prev_rb + 16 * j, 16)] = cs[j]
                        accv[pl.ds(prev_rb + 16 * j, 16)] = cp[j]

                keep = jnp.where(changed, jnp.float32(0.0),
                                 jnp.float32(1.0))
                sel = jnp.full((16,), e, jnp.int32)
                w_b = wm.at[sel].get(mode="promise_in_bounds")
                ncs, ncp = list(cs), list(cp)
                for u in range(2 * DIM // 32):
                    xi = spr[ei, pl.ds(16 * u, 16)]
                    vlo = lax.bitcast_convert_type(xi << 16, jnp.float32)
                    vhi = lax.bitcast_convert_type(
                        jnp.bitwise_and(xi, jnp.int32(-65536)),
                        jnp.float32)
                    if u < DIM // 32:
                        ncs[2 * u] = cs[2 * u] * keep + w_b * vlo
                        ncs[2 * u + 1] = cs[2 * u + 1] * keep + w_b * vhi
                    else:
                        t = u - DIM // 32
                        ncp[2 * t] = cp[2 * t] * keep + w_b * vlo
                        ncp[2 * t + 1] = cp[2 * t + 1] * keep + w_b * vhi
                cs, cp = tuple(ncs), tuple(ncp)
                prev_rb = rb_e
        return prev_rb, cs, cp

    pltpu.sync_copy(rows_hbm.at[ds8(start)], rows0)
    pltpu.sync_copy(cols_hbm.at[ds8(start)], cols0)
    fire_gathers(rows0, cols0, wi0, w0, sp0, semg0)
    pltpu.async_copy(rows_hbm.at[ds8(start + CHUNK)], rows1, semi1)
    pltpu.async_copy(cols_hbm.at[ds8(start + CHUNK)], cols1, semi1)

    def pbody(gg, carry):
        ca = start + (2 * gg) * CHUNK
        drain_w(w0, semg0)
        drain_sp(sp0, semg0)
        ra = [rows0[pl.ds(16 * h, 16)] for h in range(CHUNK // 16)]
        drain(rows1, semi1)
        drain(cols1, semi1)
        fire_gathers(rows1, cols1, wi1, w1, sp1, semg1)
        pltpu.async_copy(rows_hbm.at[ds8(ca + 2 * CHUNK)], rows0, semi0)
        pltpu.async_copy(cols_hbm.at[ds8(ca + 2 * CHUNK)], cols0, semi0)
        carry = compute(ca, ra, w0, sp0, carry)
        drain_w(w1, semg1)
        drain_sp(sp1, semg1)
        rb = [rows1[pl.ds(16 * h, 16)] for h in range(CHUNK // 16)]
        drain(rows0, semi0)
        drain(cols0, semi0)
        fire_gathers(rows0, cols0, wi0, w0, sp0, semg0)
        pltpu.async_copy(rows_hbm.at[ds8(ca + 3 * CHUNK)], rows1, semi1)
        pltpu.async_copy(cols_hbm.at[ds8(ca + 3 * CHUNK)], cols1, semi1)
        carry = compute(ca + CHUNK, rb, w1, sp1, carry)
        return carry

    zero8 = tuple(jnp.zeros((16,), jnp.float32) for _ in range(NSLICE))
    carry0 = (jnp.int32(0), zero8, zero8)
    prev_rb, cs, cp = lax.fori_loop(0, npairs, pbody, carry0)

    for j in range(NSLICE):
        accs[pl.ds(prev_rb + 16 * j, 16)] = cs[j]
        accv[pl.ds(prev_rb + 16 * j, 16)] = cp[j]

    drain_w(w0, semg0)
    drain_sp(sp0, semg0)
    drain(rows1, semi1)
    drain(cols1, semi1)

    pltpu.sync_copy(accs, aggs_hbm.at[pl.ds(wid * words, words)])
    pltpu.sync_copy(accv, aggv_hbm.at[pl.ds(wid * words, words)])


def _make_sc_edge(num_nodes, n_per_batch):
    npt = num_nodes // NTILES
    words = npt * DIM
    nlog = n_per_batch.bit_length() - 1
    assert (1 << nlog) == n_per_batch
    mesh = plsc.VectorSubcoreMesh(core_axis_name="c", subcore_axis_name="s")
    return pl.kernel(
        functools.partial(_sc_edge_body, npt, nlog),
        out_type=[jax.ShapeDtypeStruct((num_nodes * DIM,), jnp.float32),
                  jax.ShapeDtypeStruct((num_nodes * DIM,), jnp.float32)],
        mesh=mesh,
        scratch_types=[
            pltpu.VMEM((48,), jnp.int32),
            pltpu.VMEM((CHUNK, DIM), jnp.int32),
            pltpu.VMEM((CHUNK, DIM), jnp.int32),
            pltpu.VMEM((CHUNK,), jnp.float32),
            pltpu.VMEM((CHUNK,), jnp.float32),
            pltpu.VMEM((CHUNK,), jnp.int32),
            pltpu.VMEM((CHUNK,), jnp.int32),
            pltpu.VMEM((CHUNK,), jnp.int32),
            pltpu.VMEM((CHUNK,), jnp.int32),
            pltpu.VMEM((CHUNK,), jnp.int32),
            pltpu.VMEM((CHUNK,), jnp.int32),
            pltpu.VMEM((words,), jnp.float32),
            pltpu.VMEM((words,), jnp.float32),
            pltpu.SemaphoreType.DMA,
            pltpu.SemaphoreType.DMA,
            pltpu.SemaphoreType.DMA,
            pltpu.SemaphoreType.DMA,
        ],
        name="sc_edge_agg",
    )


def _row_spec(nrow, ncol):
    return pl.BlockSpec((nrow, ncol), lambda i: (i, 0))


def _full_spec(shape):
    return pl.BlockSpec(shape, lambda i: tuple(0 for _ in shape))


def _pack16(val):
    bits = lax.bitcast_convert_type(val, jnp.int32)
    cols = val.shape[1]
    out = []
    for u in range(cols // 32):
        lo = bits[:, 32 * u:32 * u + 16]
        hi = bits[:, 32 * u + 16:32 * u + 32]
        out.append(jnp.bitwise_or(
            lax.shift_right_logical(lo, 16),
            jnp.bitwise_and(hi, jnp.int32(-65536))))
    return jnp.concatenate(out, axis=1)


def _proj(x, v, qwT, qb, kwT, kb, swT, sb, vvwT, vvb, q_ref, k_ref, sp_ref):
    q_ref[...] = jnp.dot(x, qwT, preferred_element_type=jnp.float32) + qb
    k_ref[...] = jnp.dot(x, kwT, preferred_element_type=jnp.float32) + kb
    s = jnp.dot(x, swT, preferred_element_type=jnp.float32) + sb
    vv = jnp.dot(x, vvwT, preferred_element_type=jnp.float32) + vvb
    sp_ref[...] = jnp.concatenate([_pack16(s), _pack16(vv * v)], axis=1)


def _tc_init_body(tok_ref, coord_ref, embed_ref, vwT_ref, vecb_ref,
                  qwT_ref, qb_ref, kwT_ref, kb_ref, swT_ref, sb_ref,
                  vvwT_ref, vvb_ref,
                  x_ref, v_ref, q_ref, k_ref, sp_ref):
    tok = tok_ref[...]
    oh = (tok == lax.broadcasted_iota(jnp.int32, (1, DIM), 1)
          ).astype(jnp.float32)
    x = jnp.dot(oh, embed_ref[...], preferred_element_type=jnp.float32)
    c = coord_ref[...]
    vwT = vwT_ref[...]
    v = (vecb_ref[...] + c[:, 0:1] * vwT[0:1, :]
         + c[:, 1:2] * vwT[1:2, :] + c[:, 2:3] * vwT[2:3, :])
    x_ref[...], v_ref[...] = x, v
    _proj(x, v, qwT_ref[...], qb_ref[...], kwT_ref[...], kb_ref[...],
          swT_ref[...], sb_ref[...], vvwT_ref[...], vvb_ref[...],
          q_ref, k_ref, sp_ref)


def _tc_mid_body(x_ref, v_ref, aggs_ref, aggv_ref, vmwT_ref, vmb_ref,
                 qwT_ref, qb_ref, kwT_ref, kb_ref, swT_ref, sb_ref,
                 vvwT_ref, vvb_ref,
                 x2_ref, v2_ref, q_ref, k_ref, sp_ref):
    x2 = x_ref[...] + aggs_ref[...]
    v2 = (v_ref[...]
          + jnp.dot(aggv_ref[...], vmwT_ref[...],
                    preferred_element_type=jnp.float32) + vmb_ref[...])
    x2_ref[...], v2_ref[...] = x2, v2
    _proj(x2, v2, qwT_ref[...], qb_ref[...], kwT_ref[...], kb_ref[...],
          swT_ref[...], sb_ref[...], vvwT_ref[...], vvb_ref[...],
          q_ref, k_ref, sp_ref)


def _tc_fin_body(x_ref, aggs_ref, x2_ref):
    x2_ref[...] = x_ref[...] + aggs_ref[...]


def _tc_w_body(q_ref, k_ref, w_ref):
    res = lax.dot_general(
        q_ref[...], k_ref[...], (((1,), (1,)), ((), ())),
        preferred_element_type=jnp.float32)
    w_ref[...] = res.reshape(w_ref.shape)


def _tc_init(nb, tok, coords, embed, vwT, vecb, lw):
    grid = nb // BLK
    nd = jax.ShapeDtypeStruct((nb, DIM), jnp.float32)
    sp = jax.ShapeDtypeStruct((nb, DIM), jnp.int32)
    return pl.pallas_call(
        _tc_init_body,
        grid=(grid,),
        in_specs=[_row_spec(BLK, 1), _row_spec(BLK, 3),
                  _full_spec((DIM, DIM)), _full_spec((3, DIM)),
                  _full_spec((1, DIM)),
                  _full_spec((DIM, DIM)), _full_spec((1, DIM)),
                  _full_spec((DIM, DIM)), _full_spec((1, DIM)),
                  _full_spec((DIM, DIM)), _full_spec((1, DIM)),
                  _full_spec((DIM, DIM)), _full_spec((1, DIM))],
        out_specs=[_row_spec(BLK, DIM)] * 4 + [_row_spec(BLK, DIM)],
        out_shape=[nd, nd, nd, nd, sp],
        name="tc_init_proj",
    )(tok, coords, embed, vwT, vecb, *lw)


def _tc_mid(nb, x, v, aggs, aggv, vmwT, vmb, lw):
    grid = nb // BLK
    nd = jax.ShapeDtypeStruct((nb, DIM), jnp.float32)
    sp = jax.ShapeDtypeStruct((nb, DIM), jnp.int32)
    return pl.pallas_call(
        _tc_mid_body,
        grid=(grid,),
        in_specs=[_row_spec(BLK, DIM)] * 4
        + [_full_spec((DIM, DIM)), _full_spec((1, DIM)),
           _full_spec((DIM, DIM)), _full_spec((1, DIM)),
           _full_spec((DIM, DIM)), _full_spec((1, DIM)),
           _full_spec((DIM, DIM)), _full_spec((1, DIM)),
           _full_spec((DIM, DIM)), _full_spec((1, DIM))],
        out_specs=[_row_spec(BLK, DIM)] * 4 + [_row_spec(BLK, DIM)],
        out_shape=[nd, nd, nd, nd, sp],
        name="tc_mid_proj",
    )(x, v, aggs, aggv, vmwT, vmb, *lw)


def _tc_w(nb, n, q, k):
    nbat = nb // n
    grid_r = n // BLK
    return pl.pallas_call(
        _tc_w_body,
        grid=(nbat, grid_r),
        in_specs=[pl.BlockSpec((BLK, DIM), lambda b, r: (b * grid_r + r, 0)),
                  pl.BlockSpec((n, DIM), lambda b, r: (b, 0))],
        out_specs=pl.BlockSpec((BLK * n // DIM, DIM),
                               lambda b, r: (b * grid_r + r, 0)),
        out_shape=jax.ShapeDtypeStruct((nb * n // DIM, DIM), jnp.float32),
        name="tc_qkT",
    )(q, k)


def _tc_fin(nb, x, aggs):
    grid = nb // BLK
    return pl.pallas_call(
        _tc_fin_body,
        grid=(grid,),
        in_specs=[_row_spec(BLK, DIM)] * 2,
        out_specs=_row_spec(BLK, DIM),
        out_shape=jax.ShapeDtypeStruct((nb, DIM), jnp.float32),
        name="tc_fin",
    )(x, aggs)


def kernel(src_tokens, padded_coordinates, src_distance, src_edge_type,
           edge_index, params):
    b, n = src_tokens.shape
    nb = b * n
    num_layers = len(params['layers'])

    tok = src_tokens.reshape(nb, 1).astype(jnp.int32)
    coords = padded_coordinates.reshape(nb, 3).astype(jnp.float32)
    rows = edge_index[0].astype(jnp.int32)
    cols = edge_index[1].astype(jnp.int32)
    e = rows.shape[0]
    epad = ((e + 4 * CHUNK + 7) // 8) * 8 + 8
    rows_p = jnp.concatenate(
        [rows, jnp.full((epad - e,), nb - 1, jnp.int32)])
    cols_p = jnp.concatenate([cols, jnp.zeros((epad - e,), jnp.int32)])
    bounds = (jnp.arange(NTILES + 1, dtype=jnp.int32) * (nb // NTILES))
    off = jnp.searchsorted(rows, bounds, side='left').astype(jnp.int32)
    off48 = jnp.concatenate([off, jnp.full((15,), e, jnp.int32)])

    embed = params['embed']
    vwT = params['vec_w'].T
    vecb = params['vec_b'].reshape(1, DIM)

    def layer_weights(lp):
        kvw = lp['kv_w']
        kvb = lp['kv_b']
        return (lp['q_w'].T, lp['q_b'].reshape(1, DIM),
                kvw[0:DIM].T, kvb[0:DIM].reshape(1, DIM),
                kvw[DIM:2 * DIM].T, kvb[DIM:2 * DIM].reshape(1, DIM),
                kvw[2 * DIM:].T, kvb[2 * DIM:].reshape(1, DIM))

    lws = [layer_weights(lp) for lp in params['layers']]
    sc_edge = _make_sc_edge(nb, n)

    x, v, q, k, sp = _tc_init(nb, tok, coords, embed, vwT, vecb, lws[0])
    for l in range(num_layers):
        w = _tc_w(nb, n, q, k)
        aggs_f, aggv_f = sc_edge(w.reshape(nb * n), sp, rows_p, cols_p,
                                 off48)
        aggs = aggs_f.reshape(nb, DIM)
        aggv = aggv_f.reshape(nb, DIM)
        if l + 1 < num_layers:
            lp = params['layers'][l]
            x, v, q, k, sp = _tc_mid(
                nb, x, v, aggs, aggv, lp['vm_w'].T,
                lp['vm_b'].reshape(1, DIM), lws[l + 1])
        else:
            x = _tc_fin(nb, x, aggs)

    encoder_rep = x.reshape(b, n, DIM)
    padding_mask = src_tokens == 0
    return encoder_rep, padding_mask
